# baseline (speedup 1.0000x reference)
import functools

import jax
import jax.numpy as jnp
from jax import lax
from jax.experimental import pallas as pl
from jax.experimental.pallas import tpu as pltpu
from jax.experimental.pallas import tpu_sc as plsc

B = 4096
F = 26
D = 32
DD = 13
NW = 32
CHUNK = 128
IDX_PER_W = B * F // NW
NCHUNK = IDX_PER_W // CHUNK

@functools.lru_cache(maxsize=None)
def _get_sc_gather():
    mesh = plsc.VectorSubcoreMesh(core_axis_name="c", subcore_axis_name="s")

    @functools.partial(
        pl.kernel,
        mesh=mesh,
        out_type=jax.ShapeDtypeStruct((B * F, D), jnp.float32),
        scratch_types=[
            pltpu.VMEM((NCHUNK, CHUNK), jnp.int32),
            pltpu.VMEM((IDX_PER_W, D), jnp.float32),
            pltpu.SemaphoreType.DMA,
        ],
        compiler_params=pltpu.CompilerParams(use_tc_tiling_on_sc=False),
    )
    def _sc_gather(table_hbm, idx_hbm, out_hbm, idx_v, rows_v, sem):
        wid = lax.axis_index("s") * 2 + lax.axis_index("c")
        pltpu.sync_copy(idx_hbm.at[wid], idx_v)
        copies = []
        for j in range(NCHUNK):
            copies.append(
                pltpu.async_copy(
                    table_hbm.at[idx_v.at[j]],
                    rows_v.at[pl.ds(j * CHUNK, CHUNK)],
                    sem,
                )
            )
        for c in copies:
            c.wait()
        pltpu.sync_copy(rows_v, out_hbm.at[pl.ds(wid * IDX_PER_W, IDX_PER_W)])

    return _sc_gather


BB = 256


def _tc_dense_concat(sparse_ref, dense_ref, wr_ref, br_ref, out_ref):
    out_ref[:, :F, :] = sparse_ref[...]
    x = dense_ref[...]
    for j in range(DD):
        pj = jnp.dot(x, wr_ref[j], preferred_element_type=jnp.float32)
        pj = pj + br_ref[pl.ds(j, 1), :]
        out_ref[:, F + j:F + j + 1, :] = pj[:, None, :]


def kernel(sparse_inputs, dense_inputs, table, W, b):
    idx = sparse_inputs.reshape(NW, NCHUNK, CHUNK).astype(jnp.int32)
    gathered = _get_sc_gather()(table, idx)
    Wr = W.reshape(DD, D, DD).transpose(0, 2, 1)
    br = b.reshape(DD, D)
    out = pl.pallas_call(
        _tc_dense_concat,
        grid=(B // BB,),
        in_specs=[
            pl.BlockSpec((BB, F, D), lambda i: (i, 0, 0)),
            pl.BlockSpec((BB, DD), lambda i: (i, 0)),
            pl.BlockSpec((DD, DD, D), lambda i: (0, 0, 0)),
            pl.BlockSpec((DD, D), lambda i: (0, 0)),
        ],
        out_specs=pl.BlockSpec((BB, F + DD, D), lambda i: (i, 0, 0)),
        out_shape=jax.ShapeDtypeStruct((B, F + DD, D), jnp.float32),
    )(gathered.reshape(B, F, D), dense_inputs, Wr, br)
    return out


# speedup vs baseline: 1.1432x; 1.1432x over previous
import functools

import jax
import jax.numpy as jnp
from jax import lax
from jax.experimental import pallas as pl
from jax.experimental.pallas import tpu as pltpu
from jax.experimental.pallas import tpu_sc as plsc

B = 4096
F = 26
D = 32
DD = 13
NW = 32
CHUNK = 128
NIDX = B * F
IDX_PER_W = NIDX // NW
NCHUNK = IDX_PER_W // CHUNK
NOUT = (F + DD) * D


@functools.lru_cache(maxsize=None)
def _get_sc_gather():
    mesh = plsc.VectorSubcoreMesh(core_axis_name="c", subcore_axis_name="s")

    @functools.partial(
        pl.kernel,
        mesh=mesh,
        out_type=jax.ShapeDtypeStruct((NIDX, D), jnp.float32),
        scratch_types=[
            pltpu.VMEM((NCHUNK, CHUNK), jnp.int32),
            pltpu.VMEM((IDX_PER_W, D), jnp.float32),
            pltpu.SemaphoreType.DMA,
        ],
        compiler_params=pltpu.CompilerParams(use_tc_tiling_on_sc=False),
    )
    def _sc_gather(table_hbm, idx_hbm, out_hbm, idx_v, rows_v, sem):
        wid = lax.axis_index("s") * 2 + lax.axis_index("c")
        pltpu.sync_copy(idx_hbm.at[wid], idx_v)
        copies = []
        for j in range(NCHUNK):
            copies.append(
                pltpu.async_copy(
                    table_hbm.at[idx_v.at[j]],
                    rows_v.at[pl.ds(j * CHUNK, CHUNK)],
                    sem,
                )
            )
        for c in copies:
            c.wait()
        pltpu.sync_copy(rows_v, out_hbm.at[pl.ds(wid * IDX_PER_W, IDX_PER_W)])

    return _sc_gather


BB = 512


def _tc_assemble(g_ref, xt_ref, wt_ref, b_ref, out_ref):
    for f in range(F):
        out_ref[pl.ds(f * D, D), :] = g_ref[f].T
    acc = jax.lax.dot_general(
        wt_ref[...], xt_ref[...],
        (((0,), (0,)), ((), ())),
        preferred_element_type=jnp.float32,
    )
    out_ref[pl.ds(F * D, DD * D), :] = acc + b_ref[...]


def kernel(sparse_inputs, dense_inputs, table, W, b):
    idxc = sparse_inputs.T.astype(jnp.int32).reshape(NW, NCHUNK, CHUNK)
    g = _get_sc_gather()(table, idxc)
    g3 = g.reshape(F, B, D)
    xt = dense_inputs.T
    wt = W.T
    out_t = pl.pallas_call(
        _tc_assemble,
        grid=(B // BB,),
        in_specs=[
            pl.BlockSpec((F, BB, D), lambda i: (0, i, 0)),
            pl.BlockSpec((DD, BB), lambda i: (0, i)),
            pl.BlockSpec((DD, D * DD), lambda i: (0, 0)),
            pl.BlockSpec((D * DD, 1), lambda i: (0, 0)),
        ],
        out_specs=pl.BlockSpec((NOUT, BB), lambda i: (0, i)),
        out_shape=jax.ShapeDtypeStruct((NOUT, B), jnp.float32),
    )(g3, xt, wt, b.reshape(D * DD, 1))
    return out_t.reshape(F + DD, D, B).transpose(2, 0, 1)
